# baseline (speedup 1.0000x reference)
import functools

import jax
import jax.numpy as jnp
from jax import lax
from jax.experimental import pallas as pl
from jax.experimental.pallas import tpu as pltpu
from jax.experimental.pallas import tpu_sc as plsc

NC = 2
NS = 16
NW = NC * NS
LANES = 16

N_NODES = 10000
N_EDGES = 160000
D = 128

E_W = N_EDGES // NW
CHUNK = 128
N_CH = 40
E_WP = N_CH * CHUNK
N_ROWS = NW * N_CH

_mesh = plsc.VectorSubcoreMesh(core_axis_name="c", subcore_axis_name="s")


def _qk_body(x_ref, wq_ref, bq_ref, wk_ref, bk_ref, q_ref, k_ref):
    xb = x_ref[...]
    q_ref[...] = jnp.dot(xb, wq_ref[...], preferred_element_type=jnp.float32) + bq_ref[...]
    k_ref[...] = jnp.dot(xb, wk_ref[...], preferred_element_type=jnp.float32) + bk_ref[...]


def _qk_matmul(x, Wq, bq, Wk, bk):
    n, dm = x.shape
    m = Wq.shape[1]
    blk = 2000
    return pl.pallas_call(
        _qk_body,
        out_shape=(
            jax.ShapeDtypeStruct((n, m), jnp.float32),
            jax.ShapeDtypeStruct((n, m), jnp.float32),
        ),
        grid=(n // blk,),
        in_specs=[
            pl.BlockSpec((blk, dm), lambda i: (i, 0)),
            pl.BlockSpec((dm, m), lambda i: (0, 0)),
            pl.BlockSpec((1, m), lambda i: (0, 0)),
            pl.BlockSpec((dm, m), lambda i: (0, 0)),
            pl.BlockSpec((1, m), lambda i: (0, 0)),
        ],
        out_specs=(
            pl.BlockSpec((blk, m), lambda i: (i, 0)),
            pl.BlockSpec((blk, m), lambda i: (i, 0)),
        ),
    )(x, Wq, bq.reshape(1, m), Wk, bk.reshape(1, m))


@functools.partial(
    pl.kernel,
    mesh=_mesh,
    out_type=jax.ShapeDtypeStruct((N_ROWS * CHUNK * LANES,), jnp.float32),
    scratch_types=[
        pltpu.VMEM((E_WP,), jnp.int32),
        pltpu.VMEM((E_WP,), jnp.int32),
        pltpu.VMEM((CHUNK, D), jnp.float32),
        pltpu.VMEM((CHUNK, D), jnp.float32),
        pltpu.VMEM((CHUNK, D), jnp.float32),
        pltpu.VMEM((CHUNK, D), jnp.float32),
        pltpu.VMEM((CHUNK * LANES,), jnp.float32),
        pltpu.VMEM((CHUNK * LANES,), jnp.float32),
        pltpu.SemaphoreType.DMA,
        pltpu.SemaphoreType.DMA,
        pltpu.SemaphoreType.DMA,
        pltpu.SemaphoreType.DMA,
        pltpu.SemaphoreType.DMA,
        pltpu.SemaphoreType.DMA,
    ],
)
def _sc_partials(q_hbm, k_hbm, srcf_hbm, dstf_hbm, p_hbm,
                 srcall, dstall, qbA, kbA, qbB, kbB, pbA, pbB,
                 sqA, skA, sqB, skB, spA, spB):
    wid = lax.axis_index("s") * NC + lax.axis_index("c")
    pltpu.sync_copy(srcf_hbm.at[pl.ds(wid * E_WP, E_WP)], srcall)
    pltpu.sync_copy(dstf_hbm.at[pl.ds(wid * E_WP, E_WP)], dstall)

    def _start(ci, qb, kb, sq, sk):
        o = jnp.minimum(ci, N_CH - 1) * CHUNK
        pltpu.async_copy(q_hbm.at[srcall.at[pl.ds(o, CHUNK)]], qb, sq)
        pltpu.async_copy(k_hbm.at[dstall.at[pl.ds(o, CHUNK)]], kb, sk)

    def _wait_rows(qb, kb, sq, sk):
        pltpu.make_async_copy(
            q_hbm.at[srcall.at[pl.ds(0, CHUNK)]], qb, sq).wait()
        pltpu.make_async_copy(
            k_hbm.at[dstall.at[pl.ds(0, CHUNK)]], kb, sk).wait()

    def _compute(qb, kb, pb):
        def group_body(gi, _):
            goff = gi * LANES
            for el in range(LANES):
                e = goff + el
                p = qb[e, pl.ds(0, LANES)] * kb[e, pl.ds(0, LANES)]
                for j in range(1, D // LANES):
                    p = p + qb[e, pl.ds(j * LANES, LANES)] * kb[e, pl.ds(j * LANES, LANES)]
                pb[pl.ds(e * LANES, LANES)] = p
            return 0

        lax.fori_loop(0, CHUNK // LANES, group_body, 0, unroll=2)

    def _wout(ci, pb, sp):
        b = (N_CH * wid + ci) * (CHUNK * LANES)
        pltpu.async_copy(pb, p_hbm.at[pl.ds(b, CHUNK * LANES)], sp)

    def _wout_drain(pb, sp):
        pltpu.make_async_copy(
            pb, p_hbm.at[pl.ds(0, CHUNK * LANES)], sp).wait()

    _start(0, qbA, kbA, sqA, skA)

    def pair_body(cj, _):
        c0 = 2 * cj
        _start(c0 + 1, qbB, kbB, sqB, skB)
        _wait_rows(qbA, kbA, sqA, skA)

        @pl.when(cj > 0)
        def _():
            _wout_drain(pbA, spA)

        _compute(qbA, kbA, pbA)
        _wout(c0, pbA, spA)
        _start(c0 + 2, qbA, kbA, sqA, skA)
        _wait_rows(qbB, kbB, sqB, skB)

        @pl.when(cj > 0)
        def _():
            _wout_drain(pbB, spB)

        _compute(qbB, kbB, pbB)
        _wout(c0 + 1, pbB, spB)
        return 0

    lax.fori_loop(0, N_CH // 2, pair_body, 0, unroll=False)
    _wait_rows(qbA, kbA, sqA, skA)
    _wout_drain(pbA, spA)
    _wout_drain(pbB, spB)


def _exp_body(p_ref, ex_ref):
    gi = lax.broadcasted_iota(jnp.int32, (CHUNK * LANES, CHUNK), 0)
    gc = lax.broadcasted_iota(jnp.int32, (CHUNK * LANES, CHUNK), 1)
    G = jnp.where(gi // LANES == gc, 1.0, 0.0).astype(jnp.float32)
    p2 = p_ref[...].reshape(N_ROWS, CHUNK * LANES)
    s = jnp.dot(p2, G, preferred_element_type=jnp.float32,
                precision=jax.lax.Precision.HIGHEST)
    m = 0.5 * (jnp.max(s) + jnp.min(s))
    ri = lax.broadcasted_iota(jnp.int32, (N_ROWS, CHUNK), 0)
    ci = lax.broadcasted_iota(jnp.int32, (N_ROWS, CHUNK), 1)
    pad = (ri % N_CH == N_CH - 1) & (ci >= E_W - (N_CH - 1) * CHUNK)
    ex_ref[...] = jnp.where(pad, 0.0, jnp.exp(s - m))


def _tc_exp(p2):
    return pl.pallas_call(
        _exp_body,
        out_shape=jax.ShapeDtypeStruct((N_ROWS, CHUNK), jnp.float32),
    )(p2)


@functools.partial(
    pl.kernel,
    mesh=_mesh,
    out_type=(
        jax.ShapeDtypeStruct((N_NODES,), jnp.float32),
        jax.ShapeDtypeStruct((N_NODES,), jnp.float32),
    ),
    scratch_types=[
        pltpu.VMEM((N_CH, CHUNK), jnp.float32),
        pltpu.VMEM((N_CH, CHUNK), jnp.int32),
        pltpu.VMEM_SHARED((N_NODES,), jnp.float32),
    ],
)
def _sc_denoms(ex3_hbm, src3_hbm, zeros_hbm, d0_hbm, d1_hbm,
               exb, srcb, denom_sh):
    cid = lax.axis_index("c")
    sid = lax.axis_index("s")
    wid = sid * NC + cid

    @pl.when(sid == 0)
    def _():
        pltpu.sync_copy(zeros_hbm, denom_sh)

    plsc.subcore_barrier()

    pltpu.sync_copy(ex3_hbm.at[wid], exb)
    pltpu.sync_copy(src3_hbm.at[wid], srcb)

    def chunk_body(ci, _):
        pltpu.sync_copy(exb.at[ci], denom_sh.at[srcb.at[ci]], add=True)
        return 0

    lax.fori_loop(0, N_CH, chunk_body, 0, unroll=False)

    plsc.subcore_barrier()

    @pl.when((sid == 0) & (cid == 0))
    def _():
        pltpu.sync_copy(denom_sh, d0_hbm)

    @pl.when((sid == 0) & (cid == 1))
    def _():
        pltpu.sync_copy(denom_sh, d1_hbm)


@functools.partial(
    pl.kernel,
    mesh=_mesh,
    out_type=jax.ShapeDtypeStruct((NW * E_WP,), jnp.float32),
    scratch_types=[
        pltpu.VMEM((E_WP,), jnp.float32),
        pltpu.VMEM((E_WP,), jnp.int32),
        pltpu.VMEM((E_WP,), jnp.float32),
        pltpu.VMEM((E_WP,), jnp.float32),
        pltpu.VMEM((E_WP,), jnp.float32),
        pltpu.SemaphoreType.DMA,
        pltpu.SemaphoreType.DMA,
    ],
)
def _sc_normalize(exf_hbm, srcf_hbm, d0_hbm, d1_hbm, attn_hbm,
                  exb, srcb, g0, g1, ab, s0, s1):
    wid = lax.axis_index("s") * NC + lax.axis_index("c")
    base_w = wid * E_WP
    pltpu.sync_copy(exf_hbm.at[pl.ds(base_w, E_WP)], exb)
    pltpu.sync_copy(srcf_hbm.at[pl.ds(base_w, E_WP)], srcb)
    for ci in range(N_CH):
        idx = srcb.at[pl.ds(ci * CHUNK, CHUNK)]
        pltpu.async_copy(d0_hbm.at[idx], g0.at[pl.ds(ci * CHUNK, CHUNK)], s0)
        pltpu.async_copy(d1_hbm.at[idx], g1.at[pl.ds(ci * CHUNK, CHUNK)], s1)
    for _ci in range(N_CH):
        idx = srcb.at[pl.ds(0, CHUNK)]
        pltpu.make_async_copy(
            d0_hbm.at[idx], g0.at[pl.ds(0, CHUNK)], s0).wait()
        pltpu.make_async_copy(
            d1_hbm.at[idx], g1.at[pl.ds(0, CHUNK)], s1).wait()

    def group_body(gi, _):
        sl = pl.ds(gi * LANES, LANES)
        ab[sl] = exb[sl] / (g0[sl] + g1[sl])
        return 0

    lax.fori_loop(0, E_WP // LANES, group_body, 0, unroll=False)
    pltpu.sync_copy(ab, attn_hbm.at[pl.ds(base_w, E_WP)])


def kernel(x, edge_index, Wq, bq, Wk, bk):
    src = edge_index[0]
    dst = edge_index[1]
    padz = (jnp.arange(NW * (E_WP - E_W), dtype=jnp.int32)
            .reshape(NW, E_WP - E_W)) % N_NODES
    src2 = jnp.concatenate([src.reshape(NW, E_W), padz], 1)
    dst2 = jnp.concatenate([dst.reshape(NW, E_W), padz], 1)
    srcf = src2.reshape(NW * E_WP)
    dstf = dst2.reshape(NW * E_WP)
    src3 = src2.reshape(NW, N_CH, CHUNK)
    q, k = _qk_matmul(x, Wq, bq, Wk, bk)
    ex = _tc_exp(_sc_partials(q, k, srcf, dstf))
    exf = ex.reshape(NW * E_WP)
    zeros = jnp.zeros((N_NODES,), jnp.float32)
    d0, d1 = _sc_denoms(ex.reshape(NW, N_CH, CHUNK), src3, zeros)
    attn_pad = _sc_normalize(exf, srcf, d0, d1)
    attn = attn_pad.reshape(NW, E_WP)[:, :E_W].reshape(N_EDGES)
    return (x, attn)


# speedup vs baseline: 1.0113x; 1.0113x over previous
import functools

import jax
import jax.numpy as jnp
from jax import lax
from jax.experimental import pallas as pl
from jax.experimental.pallas import tpu as pltpu
from jax.experimental.pallas import tpu_sc as plsc

NC = 2
NS = 16
NW = NC * NS
LANES = 16

N_NODES = 10000
N_EDGES = 160000
D = 128

E_W = N_EDGES // NW
CHUNK = 128
N_CH = 40
E_WP = N_CH * CHUNK
N_ROWS = NW * N_CH

_mesh = plsc.VectorSubcoreMesh(core_axis_name="c", subcore_axis_name="s")


def _qk_body(x_ref, wq_ref, bq_ref, wk_ref, bk_ref, q_ref, k_ref):
    xb = x_ref[...]
    q_ref[...] = jnp.dot(xb, wq_ref[...], preferred_element_type=jnp.float32) + bq_ref[...]
    k_ref[...] = jnp.dot(xb, wk_ref[...], preferred_element_type=jnp.float32) + bk_ref[...]


def _qk_matmul(x, Wq, bq, Wk, bk):
    n, dm = x.shape
    m = Wq.shape[1]
    blk = 2000
    return pl.pallas_call(
        _qk_body,
        out_shape=(
            jax.ShapeDtypeStruct((n, m), jnp.float32),
            jax.ShapeDtypeStruct((n, m), jnp.float32),
        ),
        grid=(n // blk,),
        in_specs=[
            pl.BlockSpec((blk, dm), lambda i: (i, 0)),
            pl.BlockSpec((dm, m), lambda i: (0, 0)),
            pl.BlockSpec((1, m), lambda i: (0, 0)),
            pl.BlockSpec((dm, m), lambda i: (0, 0)),
            pl.BlockSpec((1, m), lambda i: (0, 0)),
        ],
        out_specs=(
            pl.BlockSpec((blk, m), lambda i: (i, 0)),
            pl.BlockSpec((blk, m), lambda i: (i, 0)),
        ),
    )(x, Wq, bq.reshape(1, m), Wk, bk.reshape(1, m))


@functools.partial(
    pl.kernel,
    mesh=_mesh,
    out_type=jax.ShapeDtypeStruct((N_ROWS * CHUNK * LANES,), jnp.float32),
    scratch_types=[
        pltpu.VMEM((E_WP,), jnp.int32),
        pltpu.VMEM((E_WP,), jnp.int32),
        pltpu.VMEM((CHUNK, D), jnp.float32),
        pltpu.VMEM((CHUNK, D), jnp.float32),
        pltpu.VMEM((CHUNK, D), jnp.float32),
        pltpu.VMEM((CHUNK, D), jnp.float32),
        pltpu.VMEM((CHUNK * LANES,), jnp.float32),
        pltpu.VMEM((CHUNK * LANES,), jnp.float32),
        pltpu.SemaphoreType.DMA,
        pltpu.SemaphoreType.DMA,
        pltpu.SemaphoreType.DMA,
        pltpu.SemaphoreType.DMA,
        pltpu.SemaphoreType.DMA,
        pltpu.SemaphoreType.DMA,
    ],
)
def _sc_partials(q_hbm, k_hbm, srcf_hbm, dstf_hbm, p_hbm,
                 srcall, dstall, qbA, kbA, qbB, kbB, pbA, pbB,
                 sqA, skA, sqB, skB, spA, spB):
    wid = lax.axis_index("s") * NC + lax.axis_index("c")
    pltpu.sync_copy(srcf_hbm.at[pl.ds(wid * E_WP, E_WP)], srcall)
    pltpu.sync_copy(dstf_hbm.at[pl.ds(wid * E_WP, E_WP)], dstall)

    def _start(ci, qb, kb, sq, sk):
        o = jnp.minimum(ci, N_CH - 1) * CHUNK
        pltpu.async_copy(q_hbm.at[srcall.at[pl.ds(o, CHUNK)]], qb, sq)
        pltpu.async_copy(k_hbm.at[dstall.at[pl.ds(o, CHUNK)]], kb, sk)

    def _wait_rows(qb, kb, sq, sk):
        pltpu.make_async_copy(
            q_hbm.at[srcall.at[pl.ds(0, CHUNK)]], qb, sq).wait()
        pltpu.make_async_copy(
            k_hbm.at[dstall.at[pl.ds(0, CHUNK)]], kb, sk).wait()

    def _compute(qb, kb, pb):
        def group_body(gi, _):
            goff = gi * LANES
            for el in range(LANES):
                e = goff + el
                p = qb[e, pl.ds(0, LANES)] * kb[e, pl.ds(0, LANES)]
                for j in range(1, D // LANES):
                    p = p + qb[e, pl.ds(j * LANES, LANES)] * kb[e, pl.ds(j * LANES, LANES)]
                pb[pl.ds(e * LANES, LANES)] = p
            return 0

        lax.fori_loop(0, CHUNK // LANES, group_body, 0, unroll=2)

    def _wout(ci, pb, sp):
        b = (N_CH * wid + ci) * (CHUNK * LANES)
        pltpu.async_copy(pb, p_hbm.at[pl.ds(b, CHUNK * LANES)], sp)

    def _wout_drain(pb, sp):
        pltpu.make_async_copy(
            pb, p_hbm.at[pl.ds(0, CHUNK * LANES)], sp).wait()

    _start(0, qbA, kbA, sqA, skA)

    def pair_body(cj, _):
        c0 = 2 * cj
        _start(c0 + 1, qbB, kbB, sqB, skB)
        _wait_rows(qbA, kbA, sqA, skA)

        @pl.when(cj > 0)
        def _():
            _wout_drain(pbA, spA)

        _compute(qbA, kbA, pbA)
        _wout(c0, pbA, spA)
        _start(c0 + 2, qbA, kbA, sqA, skA)
        _wait_rows(qbB, kbB, sqB, skB)

        @pl.when(cj > 0)
        def _():
            _wout_drain(pbB, spB)

        _compute(qbB, kbB, pbB)
        _wout(c0 + 1, pbB, spB)
        return 0

    lax.fori_loop(0, N_CH // 2, pair_body, 0, unroll=False)
    _wait_rows(qbA, kbA, sqA, skA)
    _wout_drain(pbA, spA)
    _wout_drain(pbB, spB)


def _exp_body(p_ref, ex_ref):
    gi = lax.broadcasted_iota(jnp.int32, (CHUNK * LANES, CHUNK), 0)
    gc = lax.broadcasted_iota(jnp.int32, (CHUNK * LANES, CHUNK), 1)
    G = jnp.where(gi // LANES == gc, 1.0, 0.0).astype(jnp.float32)
    p2 = p_ref[...].reshape(N_ROWS, CHUNK * LANES)
    s = jnp.dot(p2, G, preferred_element_type=jnp.float32,
                precision=jax.lax.Precision.HIGHEST)
    m = 0.5 * (jnp.max(s) + jnp.min(s))
    ri = lax.broadcasted_iota(jnp.int32, (N_ROWS, CHUNK), 0)
    ci = lax.broadcasted_iota(jnp.int32, (N_ROWS, CHUNK), 1)
    pad = (ri % N_CH == N_CH - 1) & (ci >= E_W - (N_CH - 1) * CHUNK)
    ex_ref[...] = jnp.where(pad, 0.0, jnp.exp(s - m))


def _tc_exp(p2):
    return pl.pallas_call(
        _exp_body,
        out_shape=jax.ShapeDtypeStruct((N_ROWS, CHUNK), jnp.float32),
    )(p2)


@functools.partial(
    pl.kernel,
    mesh=_mesh,
    out_type=(
        jax.ShapeDtypeStruct((N_NODES,), jnp.float32),
        jax.ShapeDtypeStruct((N_NODES,), jnp.float32),
    ),
    scratch_types=[
        pltpu.VMEM((N_CH, CHUNK), jnp.float32),
        pltpu.VMEM((N_CH, CHUNK), jnp.int32),
        pltpu.VMEM_SHARED((N_NODES,), jnp.float32),
        pltpu.SemaphoreType.DMA,
    ],
)
def _sc_denoms(ex3_hbm, src3_hbm, zeros_hbm, d0_hbm, d1_hbm,
               exb, srcb, denom_sh, sadd):
    cid = lax.axis_index("c")
    sid = lax.axis_index("s")
    wid = sid * NC + cid

    @pl.when(sid == 0)
    def _():
        pltpu.sync_copy(zeros_hbm, denom_sh)

    plsc.subcore_barrier()

    pltpu.sync_copy(ex3_hbm.at[wid], exb)
    pltpu.sync_copy(src3_hbm.at[wid], srcb)

    for ci in range(N_CH):
        pltpu.async_copy(exb.at[ci], denom_sh.at[srcb.at[ci]], sadd,
                         add=True)
    for _ci in range(N_CH):
        pltpu.make_async_copy(
            exb.at[0], denom_sh.at[srcb.at[0]], sadd).wait()

    plsc.subcore_barrier()

    @pl.when((sid == 0) & (cid == 0))
    def _():
        pltpu.sync_copy(denom_sh, d0_hbm)

    @pl.when((sid == 0) & (cid == 1))
    def _():
        pltpu.sync_copy(denom_sh, d1_hbm)


@functools.partial(
    pl.kernel,
    mesh=_mesh,
    out_type=jax.ShapeDtypeStruct((NW * E_WP,), jnp.float32),
    scratch_types=[
        pltpu.VMEM((E_WP,), jnp.float32),
        pltpu.VMEM((E_WP,), jnp.int32),
        pltpu.VMEM((E_WP,), jnp.float32),
        pltpu.VMEM((E_WP,), jnp.float32),
        pltpu.VMEM((E_WP,), jnp.float32),
        pltpu.SemaphoreType.DMA,
        pltpu.SemaphoreType.DMA,
    ],
)
def _sc_normalize(exf_hbm, srcf_hbm, d0_hbm, d1_hbm, attn_hbm,
                  exb, srcb, g0, g1, ab, s0, s1):
    wid = lax.axis_index("s") * NC + lax.axis_index("c")
    base_w = wid * E_WP
    pltpu.sync_copy(srcf_hbm.at[pl.ds(base_w, E_WP)], srcb)
    for ci in range(N_CH):
        idx = srcb.at[pl.ds(ci * CHUNK, CHUNK)]
        pltpu.async_copy(d0_hbm.at[idx], g0.at[pl.ds(ci * CHUNK, CHUNK)], s0)
        pltpu.async_copy(d1_hbm.at[idx], g1.at[pl.ds(ci * CHUNK, CHUNK)], s1)
    pltpu.sync_copy(exf_hbm.at[pl.ds(base_w, E_WP)], exb)
    for _ci in range(N_CH):
        idx = srcb.at[pl.ds(0, CHUNK)]
        pltpu.make_async_copy(
            d0_hbm.at[idx], g0.at[pl.ds(0, CHUNK)], s0).wait()
        pltpu.make_async_copy(
            d1_hbm.at[idx], g1.at[pl.ds(0, CHUNK)], s1).wait()

    def group_body(gi, _):
        sl = pl.ds(gi * LANES, LANES)
        ab[sl] = exb[sl] / (g0[sl] + g1[sl])
        return 0

    lax.fori_loop(0, E_WP // LANES, group_body, 0, unroll=False)
    pltpu.sync_copy(ab, attn_hbm.at[pl.ds(base_w, E_WP)])


def kernel(x, edge_index, Wq, bq, Wk, bk):
    src = edge_index[0]
    dst = edge_index[1]
    padz = (jnp.arange(NW * (E_WP - E_W), dtype=jnp.int32)
            .reshape(NW, E_WP - E_W)) % N_NODES
    src2 = jnp.concatenate([src.reshape(NW, E_W), padz], 1)
    dst2 = jnp.concatenate([dst.reshape(NW, E_W), padz], 1)
    srcf = src2.reshape(NW * E_WP)
    dstf = dst2.reshape(NW * E_WP)
    src3 = src2.reshape(NW, N_CH, CHUNK)
    q, k = _qk_matmul(x, Wq, bq, Wk, bk)
    ex = _tc_exp(_sc_partials(q, k, srcf, dstf))
    exf = ex.reshape(NW * E_WP)
    zeros = jnp.zeros((N_NODES,), jnp.float32)
    d0, d1 = _sc_denoms(ex.reshape(NW, N_CH, CHUNK), src3, zeros)
    attn_pad = _sc_normalize(exf, srcf, d0, d1)
    attn = attn_pad.reshape(NW, E_WP)[:, :E_W].reshape(N_EDGES)
    return (x, attn)
